# Initial kernel scaffold; baseline (speedup 1.0000x reference)
import jax
import jax.numpy as jnp
from jax.experimental import pallas as pl


def kernel(token_ids, table_0, table_1, table_2, table_3, table_4, table_5, table_6, table_7, fusion_w, fusion_b, rms_w):
    raise NotImplementedError("write your pallas kernel here")


# speedup vs baseline: 1.5564x; 1.5564x over previous
import dataclasses
import functools

import jax
import jax.numpy as jnp
from jax import lax
from jax.experimental import pallas as pl
from jax.experimental.pallas import tpu as pltpu
from jax.experimental.pallas import tpu_sc as plsc

_PRIMES = [251, 509, 1021, 2039, 4093, 8191, 997, 1999]
_K = 8
_D_HASH = 128
_D_MODEL = 1024
_EPS = 1e-6
_NTOK = 4 * 4096
_NROWS = _NTOK * _K
_NW = 32
_TOK_PER_W = _NTOK // _NW
_CHUNK = 128
_NCHUNK = _TOK_PER_W * _K // _CHUNK

_OFFSETS = [0]
for _p in _PRIMES[:-1]:
    _OFFSETS.append(_OFFSETS[-1] + _p)


def _sc_gather(tok_flat, stacked, primes16, off16):
    mesh = plsc.VectorSubcoreMesh(core_axis_name="c", subcore_axis_name="s")
    cp = pltpu.CompilerParams()
    if "needs_layout_passes" in pltpu.CompilerParams.__dataclass_fields__:
        cp = dataclasses.replace(cp, needs_layout_passes=False)

    @functools.partial(
        pl.kernel,
        mesh=mesh,
        compiler_params=cp,
        out_type=jax.ShapeDtypeStruct((_NROWS, _D_HASH), jnp.float32),
        scratch_types=[
            pltpu.VMEM((_TOK_PER_W,), jnp.int32),
            pltpu.VMEM((16,), jnp.int32),
            pltpu.VMEM((16,), jnp.int32),
            pltpu.VMEM((_NCHUNK, _CHUNK), jnp.int32),
            pltpu.VMEM((_CHUNK, _D_HASH), jnp.float32),
            pltpu.SemaphoreType.DMA,
        ],
    )
    def k(tok_hbm, stk_hbm, p_hbm, o_hbm, x_hbm,
          tok_v, p_v, o_v, idx_v, rows_v, sem):
        wid = lax.axis_index("s") * 2 + lax.axis_index("c")
        tbase = wid * _TOK_PER_W
        pltpu.sync_copy(tok_hbm.at[pl.ds(tbase, _TOK_PER_W)], tok_v)
        pltpu.sync_copy(p_hbm, p_v)
        pltpu.sync_copy(o_hbm, o_v)
        pv = p_v[...]
        ov = o_v[...]
        rep = lax.shift_right_logical(lax.iota(jnp.int32, 16), 3)

        @pl.loop(0, _NCHUNK)
        def _(c):
            for v in range(8):
                tvec = c * 16 + v * 2 + rep
                tok16 = plsc.load_gather(tok_v, [tvec])
                idx_v[c, pl.ds(v * 16, 16)] = ov + lax.rem(tok16, pv)

        @pl.loop(0, _NCHUNK)
        def _(g):
            pltpu.async_copy(stk_hbm.at[idx_v.at[g]], rows_v, sem).wait()
            pltpu.sync_copy(
                rows_v, x_hbm.at[pl.ds(wid * (_TOK_PER_W * _K) + g * _CHUNK,
                                       _CHUNK)])

    return k(tok_flat, stacked, primes16, off16)


def _mm_body(x_ref, w_ref, b_ref, g_ref, o_ref):
    y = jnp.dot(x_ref[...], w_ref[...],
                preferred_element_type=jnp.float32,
                precision=lax.Precision.HIGHEST)
    y = y + b_ref[...]
    ms = jnp.mean(y * y, axis=-1, keepdims=True)
    o_ref[...] = y * lax.rsqrt(ms + _EPS) * g_ref[...]


def _tc_fuse(x, fusion_w, fusion_b, rms_w, bm=1024):
    grid = (_NTOK // bm,)
    return pl.pallas_call(
        _mm_body,
        grid=grid,
        in_specs=[
            pl.BlockSpec((bm, _K * _D_HASH), lambda i: (i, 0)),
            pl.BlockSpec((_K * _D_HASH, _D_MODEL), lambda i: (0, 0)),
            pl.BlockSpec((1, _D_MODEL), lambda i: (0, 0)),
            pl.BlockSpec((1, _D_MODEL), lambda i: (0, 0)),
        ],
        out_specs=pl.BlockSpec((bm, _D_MODEL), lambda i: (i, 0)),
        out_shape=jax.ShapeDtypeStruct((_NTOK, _D_MODEL), jnp.float32),
    )(x, fusion_w, fusion_b, rms_w)


def kernel(token_ids, table_0, table_1, table_2, table_3, table_4, table_5,
           table_6, table_7, fusion_w, fusion_b, rms_w):
    tables = [table_0, table_1, table_2, table_3, table_4, table_5, table_6,
              table_7]
    stacked = jnp.concatenate(tables, axis=0)
    tok_flat = token_ids.reshape(_NTOK)
    primes16 = jnp.asarray(_PRIMES * 2, dtype=jnp.int32)
    off16 = jnp.asarray(_OFFSETS * 2, dtype=jnp.int32)
    xcat = _sc_gather(tok_flat, stacked, primes16, off16)
    x = xcat.reshape(_NTOK, _K * _D_HASH)
    y = _tc_fuse(x, fusion_w, fusion_b.reshape(1, _D_MODEL),
                 rms_w.reshape(1, _D_MODEL))
    return y.reshape(token_ids.shape[0], token_ids.shape[1], _D_MODEL)


# speedup vs baseline: 2.5408x; 1.6324x over previous
import dataclasses
import functools

import jax
import jax.numpy as jnp
from jax import lax
from jax.experimental import pallas as pl
from jax.experimental.pallas import tpu as pltpu
from jax.experimental.pallas import tpu_sc as plsc

_PRIMES = [251, 509, 1021, 2039, 4093, 8191, 997, 1999]
_K = 8
_D_HASH = 128
_D_MODEL = 1024
_EPS = 1e-6
_NTOK = 4 * 4096
_NROWS = _NTOK * _K
_NW = 32
_TOK_PER_W = _NTOK // _NW
_CHUNK = 128
_NCHUNK = _TOK_PER_W * _K // _CHUNK

_OFFSETS = [0]
for _p in _PRIMES[:-1]:
    _OFFSETS.append(_OFFSETS[-1] + _p)


def _sc_gather(tok_flat, stacked, primes16, off16):
    mesh = plsc.VectorSubcoreMesh(core_axis_name="c", subcore_axis_name="s")
    cp = pltpu.CompilerParams()
    if "needs_layout_passes" in pltpu.CompilerParams.__dataclass_fields__:
        cp = dataclasses.replace(cp, needs_layout_passes=False)

    @functools.partial(
        pl.kernel,
        mesh=mesh,
        compiler_params=cp,
        out_type=jax.ShapeDtypeStruct((_NROWS, _D_HASH), jnp.float32),
        scratch_types=[
            pltpu.VMEM((_TOK_PER_W,), jnp.int32),
            pltpu.VMEM((16,), jnp.int32),
            pltpu.VMEM((16,), jnp.int32),
            pltpu.VMEM((_NCHUNK, _CHUNK), jnp.int32),
            pltpu.VMEM((_CHUNK, _D_HASH), jnp.float32),
            pltpu.SemaphoreType.DMA,
        ],
    )
    def k(tok_hbm, stk_hbm, p_hbm, o_hbm, x_hbm,
          tok_v, p_v, o_v, idx_v, rows_v, sem):
        wid = lax.axis_index("s") * 2 + lax.axis_index("c")
        tbase = wid * _TOK_PER_W
        pltpu.sync_copy(tok_hbm.at[pl.ds(tbase, _TOK_PER_W)], tok_v)
        pltpu.sync_copy(p_hbm, p_v)
        pltpu.sync_copy(o_hbm, o_v)
        pv = p_v[...]
        ov = o_v[...]
        rep = lax.shift_right_logical(lax.iota(jnp.int32, 16), 3)

        @pl.loop(0, _NCHUNK)
        def _(c):
            for v in range(8):
                tvec = c * 16 + v * 2 + rep
                tok16 = plsc.load_gather(tok_v, [tvec])
                idx_v[c, pl.ds(v * 16, 16)] = ov + lax.rem(tok16, pv)

        @pl.loop(0, _NCHUNK)
        def _(g):
            pltpu.async_copy(stk_hbm.at[idx_v.at[g]], rows_v, sem).wait()
            pltpu.sync_copy(
                rows_v, x_hbm.at[pl.ds(wid * (_TOK_PER_W * _K) + g * _CHUNK,
                                       _CHUNK)])

    return k(tok_flat, stacked, primes16, off16)


def _mm_body(x_ref, w_ref, b_ref, g_ref, o_ref):
    y = jnp.dot(x_ref[...], w_ref[...],
                preferred_element_type=jnp.float32,
                precision=lax.Precision.DEFAULT)
    y = y + b_ref[...]
    ms = jnp.mean(y * y, axis=-1, keepdims=True)
    o_ref[...] = y * lax.rsqrt(ms + _EPS) * g_ref[...]


def _tc_fuse(x, fusion_w, fusion_b, rms_w, bm=1024):
    grid = (_NTOK // bm,)
    return pl.pallas_call(
        _mm_body,
        grid=grid,
        in_specs=[
            pl.BlockSpec((bm, _K * _D_HASH), lambda i: (i, 0)),
            pl.BlockSpec((_K * _D_HASH, _D_MODEL), lambda i: (0, 0)),
            pl.BlockSpec((1, _D_MODEL), lambda i: (0, 0)),
            pl.BlockSpec((1, _D_MODEL), lambda i: (0, 0)),
        ],
        out_specs=pl.BlockSpec((bm, _D_MODEL), lambda i: (i, 0)),
        out_shape=jax.ShapeDtypeStruct((_NTOK, _D_MODEL), jnp.float32),
    )(x, fusion_w, fusion_b, rms_w)


def kernel(token_ids, table_0, table_1, table_2, table_3, table_4, table_5,
           table_6, table_7, fusion_w, fusion_b, rms_w):
    tables = [table_0, table_1, table_2, table_3, table_4, table_5, table_6,
              table_7]
    stacked = jnp.concatenate(tables, axis=0)
    tok_flat = token_ids.reshape(_NTOK)
    primes16 = jnp.asarray(_PRIMES * 2, dtype=jnp.int32)
    off16 = jnp.asarray(_OFFSETS * 2, dtype=jnp.int32)
    xcat = _sc_gather(tok_flat, stacked, primes16, off16)
    x = xcat.reshape(_NTOK, _K * _D_HASH)
    y = _tc_fuse(x, fusion_w, fusion_b.reshape(1, _D_MODEL),
                 rms_w.reshape(1, _D_MODEL))
    return y.reshape(token_ids.shape[0], token_ids.shape[1], _D_MODEL)


# speedup vs baseline: 2.5520x; 1.0044x over previous
import dataclasses
import functools

import jax
import jax.numpy as jnp
from jax import lax
from jax.experimental import pallas as pl
from jax.experimental.pallas import tpu as pltpu
from jax.experimental.pallas import tpu_sc as plsc

_PRIMES = [251, 509, 1021, 2039, 4093, 8191, 997, 1999]
_K = 8
_D_HASH = 128
_D_MODEL = 1024
_EPS = 1e-6
_NTOK = 4 * 4096
_NROWS = _NTOK * _K
_NW = 32
_TOK_PER_W = _NTOK // _NW
_CHUNK = 128
_NCHUNK = _TOK_PER_W * _K // _CHUNK

_OFFSETS = [0]
for _p in _PRIMES[:-1]:
    _OFFSETS.append(_OFFSETS[-1] + _p)


def _sc_gather(tok_flat, stacked, primes16, off16):
    mesh = plsc.VectorSubcoreMesh(core_axis_name="c", subcore_axis_name="s")
    cp = pltpu.CompilerParams()
    if "needs_layout_passes" in pltpu.CompilerParams.__dataclass_fields__:
        cp = dataclasses.replace(cp, needs_layout_passes=False)

    @functools.partial(
        pl.kernel,
        mesh=mesh,
        compiler_params=cp,
        out_type=jax.ShapeDtypeStruct((_NROWS, _D_HASH), jnp.float32),
        scratch_types=[
            pltpu.VMEM((_TOK_PER_W,), jnp.int32),
            pltpu.VMEM((16,), jnp.int32),
            pltpu.VMEM((16,), jnp.int32),
            pltpu.VMEM((_NCHUNK, _CHUNK), jnp.int32),
            pltpu.VMEM((_CHUNK, _D_HASH), jnp.float32),
            pltpu.SemaphoreType.DMA,
        ],
    )
    def k(tok_hbm, stk_hbm, p_hbm, o_hbm, x_hbm,
          tok_v, p_v, o_v, idx_v, rows_v, sem):
        wid = lax.axis_index("s") * 2 + lax.axis_index("c")
        tbase = wid * _TOK_PER_W
        pltpu.sync_copy(tok_hbm.at[pl.ds(tbase, _TOK_PER_W)], tok_v)
        pltpu.sync_copy(p_hbm, p_v)
        pltpu.sync_copy(o_hbm, o_v)
        pv = p_v[...]
        ov = o_v[...]
        rep = lax.shift_right_logical(lax.iota(jnp.int32, 16), 3)

        @pl.loop(0, _NCHUNK)
        def _(c):
            for v in range(8):
                tvec = c * 16 + v * 2 + rep
                tok16 = plsc.load_gather(tok_v, [tvec])
                idx_v[c, pl.ds(v * 16, 16)] = ov + lax.rem(tok16, pv)

        @pl.loop(0, _NCHUNK)
        def _(g):
            pltpu.async_copy(stk_hbm.at[idx_v.at[g]], rows_v, sem).wait()
            pltpu.sync_copy(
                rows_v, x_hbm.at[pl.ds(wid * (_TOK_PER_W * _K) + g * _CHUNK,
                                       _CHUNK)])

    return k(tok_flat, stacked, primes16, off16)


def _mm_body(x_ref, w_ref, b_ref, g_ref, o_ref):
    y = jnp.dot(x_ref[...], w_ref[...],
                preferred_element_type=jnp.float32,
                precision=lax.Precision.DEFAULT)
    y = y + b_ref[...]
    ms = jnp.mean(y * y, axis=-1, keepdims=True)
    o_ref[...] = y * lax.rsqrt(ms + _EPS) * g_ref[...]


def _tc_fuse(x, fusion_w, fusion_b, rms_w, bm=2048):
    grid = (_NTOK // bm,)
    return pl.pallas_call(
        _mm_body,
        grid=grid,
        in_specs=[
            pl.BlockSpec((bm, _K * _D_HASH), lambda i: (i, 0)),
            pl.BlockSpec((_K * _D_HASH, _D_MODEL), lambda i: (0, 0)),
            pl.BlockSpec((1, _D_MODEL), lambda i: (0, 0)),
            pl.BlockSpec((1, _D_MODEL), lambda i: (0, 0)),
        ],
        out_specs=pl.BlockSpec((bm, _D_MODEL), lambda i: (i, 0)),
        out_shape=jax.ShapeDtypeStruct((_NTOK, _D_MODEL), jnp.float32),
    )(x, fusion_w, fusion_b, rms_w)


def kernel(token_ids, table_0, table_1, table_2, table_3, table_4, table_5,
           table_6, table_7, fusion_w, fusion_b, rms_w):
    tables = [table_0, table_1, table_2, table_3, table_4, table_5, table_6,
              table_7]
    stacked = jnp.concatenate(tables, axis=0)
    tok_flat = token_ids.reshape(_NTOK)
    primes16 = jnp.asarray(_PRIMES * 2, dtype=jnp.int32)
    off16 = jnp.asarray(_OFFSETS * 2, dtype=jnp.int32)
    xcat = _sc_gather(tok_flat, stacked, primes16, off16)
    x = xcat.reshape(_NTOK, _K * _D_HASH)
    y = _tc_fuse(x, fusion_w, fusion_b.reshape(1, _D_MODEL),
                 rms_w.reshape(1, _D_MODEL))
    return y.reshape(token_ids.shape[0], token_ids.shape[1], _D_MODEL)


# speedup vs baseline: 3.4014x; 1.3328x over previous
import dataclasses
import functools

import jax
import jax.numpy as jnp
from jax import lax
from jax.experimental import pallas as pl
from jax.experimental.pallas import tpu as pltpu
from jax.experimental.pallas import tpu_sc as plsc

_PRIMES = [251, 509, 1021, 2039, 4093, 8191, 997, 1999]
_K = 8
_D_HASH = 128
_D_MODEL = 1024
_EPS = 1e-6
_NTOK = 4 * 4096
_NROWS = _NTOK * _K
_NW = 32
_TOK_PER_W = _NTOK // _NW
_CHUNK = 128
_NCHUNK = _TOK_PER_W * _K // _CHUNK

_OFFSETS = [0]
for _p in _PRIMES[:-1]:
    _OFFSETS.append(_OFFSETS[-1] + _p)


def _sc_gather(tok_flat, stacked, primes16, off16):
    mesh = plsc.VectorSubcoreMesh(core_axis_name="c", subcore_axis_name="s")
    cp = pltpu.CompilerParams()
    if "needs_layout_passes" in pltpu.CompilerParams.__dataclass_fields__:
        cp = dataclasses.replace(cp, needs_layout_passes=False)

    @functools.partial(
        pl.kernel,
        mesh=mesh,
        compiler_params=cp,
        out_type=jax.ShapeDtypeStruct((_NROWS, _D_HASH), jnp.float32),
        scratch_types=[
            pltpu.VMEM((_TOK_PER_W,), jnp.int32),
            pltpu.VMEM((16,), jnp.int32),
            pltpu.VMEM((16,), jnp.int32),
            pltpu.VMEM((_NCHUNK, _CHUNK), jnp.int32),
            pltpu.VMEM((_CHUNK, _D_HASH), jnp.float32),
            pltpu.SemaphoreType.DMA,
        ],
    )
    def k(tok_hbm, stk_hbm, p_hbm, o_hbm, x_hbm,
          tok_v, p_v, o_v, idx_v, rows_v, sem):
        wid = lax.axis_index("s") * 2 + lax.axis_index("c")
        tbase = wid * _TOK_PER_W
        pltpu.sync_copy(tok_hbm.at[pl.ds(tbase, _TOK_PER_W)], tok_v)
        pltpu.sync_copy(p_hbm, p_v)
        pltpu.sync_copy(o_hbm, o_v)
        pv = p_v[...]
        ov = o_v[...]
        rep = lax.shift_right_logical(lax.iota(jnp.int32, 16), 3)

        @pl.loop(0, _NCHUNK)
        def _(c):
            for v in range(8):
                tvec = c * 16 + v * 2 + rep
                tok16 = plsc.load_gather(tok_v, [tvec])
                idx_v[c, pl.ds(v * 16, 16)] = ov + lax.rem(tok16, pv)

        @pl.loop(0, _NCHUNK)
        def _(g):
            pltpu.async_copy(stk_hbm.at[idx_v.at[g]], rows_v, sem).wait()
            pltpu.sync_copy(
                rows_v, x_hbm.at[pl.ds(wid * (_TOK_PER_W * _K) + g * _CHUNK,
                                       _CHUNK)])

    return k(tok_flat, stacked, primes16, off16)


def _mm_body(x_ref, w_ref, b_ref, g_ref, o_ref):
    bm = x_ref.shape[0] // _K
    y = jnp.dot(x_ref[...].reshape(bm, _K * _D_HASH), w_ref[...],
                preferred_element_type=jnp.float32,
                precision=lax.Precision.DEFAULT)
    y = y + b_ref[...]
    ms = jnp.mean(y * y, axis=-1, keepdims=True)
    o_ref[...] = y * lax.rsqrt(ms + _EPS) * g_ref[...]


def _tc_fuse(x, fusion_w, fusion_b, rms_w, bm=2048):
    grid = (_NTOK // bm,)
    return pl.pallas_call(
        _mm_body,
        grid=grid,
        in_specs=[
            pl.BlockSpec((bm * _K, _D_HASH), lambda i: (i, 0)),
            pl.BlockSpec((_K * _D_HASH, _D_MODEL), lambda i: (0, 0)),
            pl.BlockSpec((1, _D_MODEL), lambda i: (0, 0)),
            pl.BlockSpec((1, _D_MODEL), lambda i: (0, 0)),
        ],
        out_specs=pl.BlockSpec((bm, _D_MODEL), lambda i: (i, 0)),
        out_shape=jax.ShapeDtypeStruct((_NTOK, _D_MODEL), jnp.float32),
    )(x, fusion_w, fusion_b, rms_w)


def kernel(token_ids, table_0, table_1, table_2, table_3, table_4, table_5,
           table_6, table_7, fusion_w, fusion_b, rms_w):
    tables = [table_0, table_1, table_2, table_3, table_4, table_5, table_6,
              table_7]
    stacked = jnp.concatenate(tables, axis=0)
    tok_flat = token_ids.reshape(_NTOK)
    primes16 = jnp.asarray(_PRIMES * 2, dtype=jnp.int32)
    off16 = jnp.asarray(_OFFSETS * 2, dtype=jnp.int32)
    xcat = _sc_gather(tok_flat, stacked, primes16, off16)
    y = _tc_fuse(xcat, fusion_w, fusion_b.reshape(1, _D_MODEL),
                 rms_w.reshape(1, _D_MODEL))
    return y.reshape(token_ids.shape[0], token_ids.shape[1], _D_MODEL)
